# baseline (speedup 1.0000x reference)
import functools

import jax
import jax.numpy as jnp
from jax import lax
from jax.experimental import pallas as pl
from jax.experimental.pallas import tpu as pltpu
from jax.experimental.pallas import tpu_sc as plsc

ROWS, COLS = 4096, 200
NBREAKS = 34
NBINS = NBREAKS - 1
N = ROWS * COLS
NW = 32
PER_W = N // NW
E = 1600
NCH = PER_W // E
OUTW = E * NBINS
L = 16
GROUPS = E // L
BPAD = 48


@functools.partial(
    pl.kernel,
    out_type=jax.ShapeDtypeStruct((N * NBINS,), jnp.float32),
    mesh=plsc.VectorSubcoreMesh(
        core_axis_name="c", subcore_axis_name="s", num_cores=2, num_subcores=16
    ),
    compiler_params=pltpu.CompilerParams(needs_layout_passes=False),
    scratch_types=[
        pltpu.VMEM((OUTW,), jnp.float32),
        pltpu.VMEM((OUTW,), jnp.float32),
        pltpu.VMEM((E,), jnp.float32),
        pltpu.VMEM((E,), jnp.float32),
        pltpu.VMEM((E,), jnp.int32),
        pltpu.VMEM((E,), jnp.int32),
        pltpu.VMEM((BPAD,), jnp.float32),
        pltpu.SemaphoreType.DMA,
        pltpu.SemaphoreType.DMA,
        pltpu.SemaphoreType.DMA,
        pltpu.SemaphoreType.DMA,
    ],
)
def _sc_onehot(x_hbm, breaks_hbm, out_hbm,
               out0, out1, in0, in1, pos0, pos1, brk,
               sin0, sin1, sout0, sout1):
    wid = lax.axis_index("s") * 2 + lax.axis_index("c")
    ebase = wid * PER_W
    obase = wid * (PER_W * NBINS)

    pltpu.sync_copy(breaks_hbm, brk)

    lane33 = lax.iota(jnp.int32, L) * NBINS
    ones = jnp.full((L,), 1.0, jnp.float32)
    zeros = jnp.zeros((L,), jnp.float32)

    def zinit(j, c):
        out0[pl.ds(j * L, L)] = zeros
        out1[pl.ds(j * L, L)] = zeros
        return c

    lax.fori_loop(0, OUTW // L, zinit, 0)

    pltpu.async_copy(x_hbm.at[pl.ds(ebase, E)], in0, sin0)
    pltpu.async_copy(x_hbm.at[pl.ds(ebase + E, E)], in1, sin1)

    bufs = ((out0, in0, pos0, sin0, sout0), (out1, in1, pos1, sin1, sout1))

    def chunk_pair(p, c):
        for b in range(2):
            outb, inb, posb, sinb, soutb = bufs[b]
            i = 2 * p + b

            pltpu.make_async_copy(x_hbm.at[pl.ds(0, E)], inb, sinb).wait()

            @pl.when(i >= 2)
            def _():
                pltpu.make_async_copy(
                    outb, out_hbm.at[pl.ds(0, OUTW)], soutb
                ).wait()

                def unz(j, cc):
                    old = posb[pl.ds(j * L, L)]
                    plsc.store_scatter(outb, [old], zeros)
                    return cc

                lax.fori_loop(0, GROUPS, unz, 0)

            def grp(j, cc):
                x = inb[pl.ds(j * L, L)]
                g = (x * jnp.float32(NBREAKS - 3)).astype(jnp.int32) + 1
                g = jnp.clip(g, 0, NBINS - 1)
                for _ in range(2):
                    bu = plsc.load_gather(brk, [g + 1])
                    g = jnp.where(bu <= x, g + 1, g)
                    bl = plsc.load_gather(brk, [g])
                    g = jnp.where(bl > x, g - 1, g)
                pos = lane33 + (j * (L * NBINS) + g)
                plsc.store_scatter(outb, [pos], ones)
                posb[pl.ds(j * L, L)] = pos
                return cc

            lax.fori_loop(0, GROUPS, grp, 0)

            pltpu.async_copy(
                outb, out_hbm.at[pl.ds(obase + i * OUTW, OUTW)], soutb
            )

            @pl.when(i + 2 < NCH)
            def _():
                pltpu.async_copy(
                    x_hbm.at[pl.ds(ebase + (i + 2) * E, E)], inb, sinb
                )

        return c

    lax.fori_loop(0, NCH // 2, chunk_pair, 0)

    pltpu.make_async_copy(out0, out_hbm.at[pl.ds(0, OUTW)], sout0).wait()
    pltpu.make_async_copy(out1, out_hbm.at[pl.ds(0, OUTW)], sout1).wait()


def kernel(input, breaks):
    pad = jnp.full((BPAD - NBREAKS,), jnp.inf, dtype=breaks.dtype)
    breaks_padded = jnp.concatenate([breaks, pad])
    flat = _sc_onehot(jnp.reshape(input, (N,)), breaks_padded)
    return jnp.reshape(flat, (ROWS, COLS, NBINS))


# speedup vs baseline: 1.0656x; 1.0656x over previous
import functools

import jax
import jax.numpy as jnp
from jax import lax
from jax.experimental import pallas as pl
from jax.experimental.pallas import tpu as pltpu
from jax.experimental.pallas import tpu_sc as plsc

ROWS, COLS = 4096, 200
NBREAKS = 34
NBINS = NBREAKS - 1
N = ROWS * COLS
NW = 32
PER_W = N // NW
E = 1600
NCH = PER_W // E
OUTW = E * NBINS
L = 16
GROUPS = E // L
BPAD = 48


@functools.partial(
    pl.kernel,
    out_type=jax.ShapeDtypeStruct((N * NBINS,), jnp.float32),
    mesh=plsc.VectorSubcoreMesh(
        core_axis_name="c", subcore_axis_name="s", num_cores=2, num_subcores=16
    ),
    compiler_params=pltpu.CompilerParams(needs_layout_passes=False),
    scratch_types=[
        pltpu.VMEM((OUTW,), jnp.float32),
        pltpu.VMEM((OUTW,), jnp.float32),
        pltpu.VMEM((E,), jnp.float32),
        pltpu.VMEM((E,), jnp.float32),
        pltpu.VMEM((E,), jnp.int32),
        pltpu.VMEM((E,), jnp.int32),
        pltpu.VMEM((BPAD,), jnp.float32),
        pltpu.SemaphoreType.DMA,
        pltpu.SemaphoreType.DMA,
        pltpu.SemaphoreType.DMA,
        pltpu.SemaphoreType.DMA,
    ],
)
def _sc_onehot(x_hbm, breaks_hbm, out_hbm,
               out0, out1, in0, in1, pos0, pos1, brk,
               sin0, sin1, sout0, sout1):
    wid = lax.axis_index("s") * 2 + lax.axis_index("c")
    ebase = wid * PER_W
    obase = wid * (PER_W * NBINS)

    pltpu.sync_copy(breaks_hbm, brk)

    lane33 = lax.iota(jnp.int32, L) * NBINS
    ones = jnp.full((L,), 1.0, jnp.float32)
    zeros = jnp.zeros((L,), jnp.float32)

    @plsc.parallel_loop(0, OUTW // L, unroll=4)
    def _(j):
        out0[pl.ds(j * L, L)] = zeros
        out1[pl.ds(j * L, L)] = zeros

    pltpu.async_copy(x_hbm.at[pl.ds(ebase, E)], in0, sin0)
    pltpu.async_copy(x_hbm.at[pl.ds(ebase + E, E)], in1, sin1)

    bufs = ((out0, in0, pos0, sin0, sout0), (out1, in1, pos1, sin1, sout1))

    def chunk_pair(p, c):
        for b in range(2):
            outb, inb, posb, sinb, soutb = bufs[b]
            i = 2 * p + b

            pltpu.make_async_copy(x_hbm.at[pl.ds(0, E)], inb, sinb).wait()

            @pl.when(i >= 2)
            def _():
                pltpu.make_async_copy(
                    outb, out_hbm.at[pl.ds(0, OUTW)], soutb
                ).wait()

                @plsc.parallel_loop(0, GROUPS, unroll=4)
                def _(j):
                    old = posb[pl.ds(j * L, L)]
                    plsc.store_scatter(outb, [old], zeros)

            @plsc.parallel_loop(0, GROUPS, unroll=4)
            def _(j):
                x = inb[pl.ds(j * L, L)]
                g = (x * jnp.float32(NBREAKS - 3)).astype(jnp.int32) + 1
                g = jnp.clip(g, 0, NBINS - 1)
                bu = plsc.load_gather(brk, [g + 1])
                bl = plsc.load_gather(brk, [g])
                g = g + (bu <= x).astype(jnp.int32) - (bl > x).astype(jnp.int32)
                pos = lane33 + (j * (L * NBINS) + g)
                plsc.store_scatter(outb, [pos], ones)
                posb[pl.ds(j * L, L)] = pos

            pltpu.async_copy(
                outb, out_hbm.at[pl.ds(obase + i * OUTW, OUTW)], soutb
            )

            @pl.when(i + 2 < NCH)
            def _():
                pltpu.async_copy(
                    x_hbm.at[pl.ds(ebase + (i + 2) * E, E)], inb, sinb
                )

        return c

    lax.fori_loop(0, NCH // 2, chunk_pair, 0)

    pltpu.make_async_copy(out0, out_hbm.at[pl.ds(0, OUTW)], sout0).wait()
    pltpu.make_async_copy(out1, out_hbm.at[pl.ds(0, OUTW)], sout1).wait()


def kernel(input, breaks):
    pad = jnp.full((BPAD - NBREAKS,), jnp.inf, dtype=breaks.dtype)
    breaks_padded = jnp.concatenate([breaks, pad])
    flat = _sc_onehot(jnp.reshape(input, (N,)), breaks_padded)
    return jnp.reshape(flat, (ROWS, COLS, NBINS))


# speedup vs baseline: 1.3294x; 1.2476x over previous
import functools

import jax
import jax.numpy as jnp
from jax import lax
from jax.experimental import pallas as pl
from jax.experimental.pallas import tpu as pltpu
from jax.experimental.pallas import tpu_sc as plsc

ROWS, COLS = 4096, 200
NBREAKS = 34
NBINS = NBREAKS - 1
N = ROWS * COLS
NW = 32
PER_W = N // NW
E = 800
NCH = PER_W // E
OUTW = E * NBINS
L = 16
GROUPS = E // L
RPC = E // COLS
BPAD = 48


@functools.partial(
    pl.kernel,
    out_type=jax.ShapeDtypeStruct((ROWS, COLS, NBINS), jnp.float32),
    mesh=plsc.VectorSubcoreMesh(
        core_axis_name="c", subcore_axis_name="s", num_cores=2, num_subcores=16
    ),
    compiler_params=pltpu.CompilerParams(needs_layout_passes=False, use_tc_tiling_on_sc=False),
    scratch_types=[
        pltpu.VMEM((RPC, COLS, NBINS), jnp.float32),
        pltpu.VMEM((RPC, COLS, NBINS), jnp.float32),
        pltpu.VMEM((E,), jnp.float32),
        pltpu.VMEM((E,), jnp.float32),
        pltpu.VMEM((E,), jnp.int32),
        pltpu.VMEM((E,), jnp.int32),
        pltpu.VMEM((BPAD,), jnp.float32),
        pltpu.SemaphoreType.DMA,
        pltpu.SemaphoreType.DMA,
        pltpu.SemaphoreType.DMA,
        pltpu.SemaphoreType.DMA,
    ],
)
def _sc_onehot(x_hbm, breaks_hbm, out3d_hbm,
               out0, out1, in0, in1, pos0, pos1, brk,
               sin0, sin1, sout0, sout1):
    wid = lax.axis_index("s") * 2 + lax.axis_index("c")
    ebase = wid * PER_W
    rbase = wid * (PER_W // COLS)

    pltpu.sync_copy(breaks_hbm, brk)

    lane = lax.iota(jnp.int32, L)
    ones = jnp.full((L,), 1.0, jnp.float32)
    zeros = jnp.zeros((L,), jnp.float32)

    @plsc.parallel_loop(0, OUTW // L, unroll=4)
    def _(j):
        w = j * L + lane
        t = w // NBINS
        k = w - t * NBINS
        r = t // COLS
        f = t - r * COLS
        plsc.store_scatter(out0, [r, f, k], zeros)
        plsc.store_scatter(out1, [r, f, k], zeros)

    pltpu.async_copy(x_hbm.at[pl.ds(ebase, E)], in0, sin0)
    pltpu.async_copy(x_hbm.at[pl.ds(ebase + E, E)], in1, sin1)

    bufs = ((out0, in0, pos0, sin0, sout0), (out1, in1, pos1, sin1, sout1))

    def chunk_pair(p, c):
        for b in range(2):
            outb, inb, posb, sinb, soutb = bufs[b]
            i = 2 * p + b

            pltpu.make_async_copy(x_hbm.at[pl.ds(0, E)], inb, sinb).wait()

            @pl.when(i >= 2)
            def _():
                pltpu.make_async_copy(
                    outb, out3d_hbm.at[pl.ds(0, RPC)], soutb
                ).wait()

                @plsc.parallel_loop(0, GROUPS, unroll=4)
                def _(j):
                    e = j * L + lane
                    r = e // COLS
                    f = e - r * COLS
                    g = posb[pl.ds(j * L, L)]
                    plsc.store_scatter(outb, [r, f, g], zeros)

            @plsc.parallel_loop(0, GROUPS, unroll=4)
            def _(j):
                x = inb[pl.ds(j * L, L)]
                g = (x * jnp.float32(NBREAKS - 3)).astype(jnp.int32) + 1
                g = jnp.clip(g, 0, NBINS - 1)
                bu = plsc.load_gather(brk, [g + 1])
                bl = plsc.load_gather(brk, [g])
                g = g + (bu <= x).astype(jnp.int32) - (bl > x).astype(jnp.int32)
                e = j * L + lane
                r = e // COLS
                f = e - r * COLS
                plsc.store_scatter(outb, [r, f, g], ones)
                posb[pl.ds(j * L, L)] = g

            pltpu.async_copy(
                outb, out3d_hbm.at[pl.ds(rbase + i * RPC, RPC)], soutb
            )

            @pl.when(i + 2 < NCH)
            def _():
                pltpu.async_copy(
                    x_hbm.at[pl.ds(ebase + (i + 2) * E, E)], inb, sinb
                )

        return c

    lax.fori_loop(0, NCH // 2, chunk_pair, 0)

    pltpu.make_async_copy(out0, out3d_hbm.at[pl.ds(0, RPC)], sout0).wait()
    pltpu.make_async_copy(out1, out3d_hbm.at[pl.ds(0, RPC)], sout1).wait()


def kernel(input, breaks):
    pad = jnp.full((BPAD - NBREAKS,), jnp.inf, dtype=breaks.dtype)
    breaks_padded = jnp.concatenate([breaks, pad])
    return _sc_onehot(jnp.reshape(input, (N,)), breaks_padded)


# speedup vs baseline: 13.6180x; 10.2436x over previous
import functools

import jax
import jax.numpy as jnp
from jax import lax
from jax.experimental import pallas as pl
from jax.experimental.pallas import tpu as pltpu
from jax.experimental.pallas import tpu_sc as plsc

ROWS, COLS = 4096, 200
NBREAKS = 34
NBINS = NBREAKS - 1
N = ROWS * COLS
NW = 32
TR = ROWS // 128
TF = COLS // 8
PER_W = N // NW
CHW = NBINS * 8 * 128
L = 16
GROUPS = 8 * 128 // L
BPAD = 48


@functools.partial(
    pl.kernel,
    out_type=jax.ShapeDtypeStruct((NBINS, TF, TR, 8, 128), jnp.float32),
    mesh=plsc.VectorSubcoreMesh(
        core_axis_name="c", subcore_axis_name="s", num_cores=2, num_subcores=16
    ),
    compiler_params=pltpu.CompilerParams(
        needs_layout_passes=False, use_tc_tiling_on_sc=False
    ),
    scratch_types=[
        pltpu.VMEM((NBINS, 1, 1, 8, 128), jnp.float32),
        pltpu.VMEM((NBINS, 1, 1, 8, 128), jnp.float32),
        pltpu.VMEM((1, PER_W), jnp.float32),
        pltpu.VMEM((8 * 128,), jnp.int32),
        pltpu.VMEM((8 * 128,), jnp.int32),
        pltpu.VMEM((BPAD,), jnp.float32),
        pltpu.SemaphoreType.DMA,
        pltpu.SemaphoreType.DMA,
        pltpu.SemaphoreType.DMA,
    ],
)
def _sc_onehot(xt_hbm, breaks_hbm, out5_hbm,
               st0, st1, inb, pos0, pos1, brk,
               sin, sout0, sout1):
    wid = lax.axis_index("s") * 2 + lax.axis_index("c")

    pltpu.sync_copy(breaks_hbm, brk)
    pltpu.async_copy(xt_hbm.at[pl.ds(wid, 1)], inb, sin)

    lane = lax.iota(jnp.int32, L)
    ones = jnp.full((L,), 1.0, jnp.float32)
    zeros = jnp.zeros((L,), jnp.float32)
    zi = jnp.zeros((L,), jnp.int32)

    @plsc.parallel_loop(0, CHW // L, unroll=4)
    def _(j):
        w = j * L + lane
        k = w >> 10
        rem = w & 1023
        fl = rem >> 7
        rl = rem & 127
        plsc.store_scatter(st0, [k, zi, zi, fl, rl], zeros)
        plsc.store_scatter(st1, [k, zi, zi, fl, rl], zeros)

    pltpu.make_async_copy(xt_hbm.at[pl.ds(0, 1)], inb, sin).wait()

    def unzero(stb, posb):
        @plsc.parallel_loop(0, GROUPS, unroll=4)
        def _(j):
            fl = (j >> 3) + zi
            rl = (j & 7) * L + lane
            g = posb[pl.ds(j * L, L)]
            plsc.store_scatter(stb, [g, zi, zi, fl, rl], zeros)

    def fill(stb, posb, tf):
        @plsc.parallel_loop(0, GROUPS, unroll=4)
        def _(j):
            x = inb[0, pl.ds(tf * (8 * 128) + j * L, L)]
            g = (x * jnp.float32(NBREAKS - 3)).astype(jnp.int32) + 1
            g = jnp.clip(g, 0, NBINS - 1)
            bu = plsc.load_gather(brk, [g + 1])
            bl = plsc.load_gather(brk, [g])
            g = g + (bu <= x).astype(jnp.int32) - (bl > x).astype(jnp.int32)
            fl = (j >> 3) + zi
            rl = (j & 7) * L + lane
            plsc.store_scatter(stb, [g, zi, zi, fl, rl], ones)
            posb[pl.ds(j * L, L)] = g

    bufs = ((st0, pos0, sout0), (st1, pos1, sout1))

    def chunk_pair(p, c):
        for b in range(2):
            stb, posb, soutb = bufs[b]
            tf = 2 * p + b

            @pl.when(tf >= 2)
            def _():
                pltpu.make_async_copy(
                    stb, out5_hbm.at[:, pl.ds(0, 1), pl.ds(0, 1)], soutb
                ).wait()
                unzero(stb, posb)

            fill(stb, posb, tf)

            pltpu.async_copy(
                stb, out5_hbm.at[:, pl.ds(tf, 1), pl.ds(wid, 1)], soutb
            )

        return c

    lax.fori_loop(0, TF // 2, chunk_pair, 0)

    pltpu.make_async_copy(
        st0, out5_hbm.at[:, pl.ds(0, 1), pl.ds(0, 1)], sout0
    ).wait()
    unzero(st0, pos0)
    fill(st0, pos0, TF - 1)
    pltpu.async_copy(
        st0, out5_hbm.at[:, pl.ds(TF - 1, 1), pl.ds(wid, 1)], sout0
    )
    pltpu.make_async_copy(
        st0, out5_hbm.at[:, pl.ds(0, 1), pl.ds(0, 1)], sout0
    ).wait()
    pltpu.make_async_copy(
        st1, out5_hbm.at[:, pl.ds(0, 1), pl.ds(0, 1)], sout1
    ).wait()


def kernel(input, breaks):
    pad = jnp.full((BPAD - NBREAKS,), jnp.inf, dtype=breaks.dtype)
    breaks_padded = jnp.concatenate([breaks, pad])
    xt = (
        input.reshape(TR, 128, TF, 8)
        .transpose(0, 2, 3, 1)
        .reshape(TR, PER_W)
    )
    out5 = _sc_onehot(xt, breaks_padded)
    return out5.transpose(2, 4, 1, 3, 0).reshape(ROWS, COLS, NBINS)


# speedup vs baseline: 13.9451x; 1.0240x over previous
import functools

import jax
import jax.numpy as jnp
from jax import lax
from jax.experimental import pallas as pl
from jax.experimental.pallas import tpu as pltpu
from jax.experimental.pallas import tpu_sc as plsc

ROWS, COLS = 4096, 200
NBREAKS = 34
NBINS = NBREAKS - 1
N = ROWS * COLS
NW = 32
TR = ROWS // 128
TF = COLS // 8
PER_W = N // NW
CHW = NBINS * 8 * 128
L = 16
GROUPS = 8 * 128 // L
BPAD = 48


@functools.partial(
    pl.kernel,
    out_type=jax.ShapeDtypeStruct((NBINS, TF, TR, 8, 128), jnp.float32),
    mesh=plsc.VectorSubcoreMesh(
        core_axis_name="c", subcore_axis_name="s", num_cores=2, num_subcores=16
    ),
    compiler_params=pltpu.CompilerParams(
        needs_layout_passes=False,
        use_tc_tiling_on_sc=False,
        disable_bounds_checks=True,
        disable_semaphore_checks=True,
    ),
    scratch_types=[
        pltpu.VMEM((NBINS, 1, 1, 8, 128), jnp.float32),
        pltpu.VMEM((NBINS, 1, 1, 8, 128), jnp.float32),
        pltpu.VMEM((TF, 1, 8, 128), jnp.float32),
        pltpu.VMEM((8 * 128,), jnp.int32),
        pltpu.VMEM((8 * 128,), jnp.int32),
        pltpu.VMEM((BPAD,), jnp.float32),
        pltpu.SemaphoreType.DMA,
        pltpu.SemaphoreType.DMA,
        pltpu.SemaphoreType.DMA,
    ],
)
def _sc_onehot(xt_hbm, breaks_hbm, out5_hbm,
               st0, st1, inb, pos0, pos1, brk,
               sin, sout0, sout1):
    wid = lax.axis_index("s") * 2 + lax.axis_index("c")

    pltpu.sync_copy(breaks_hbm, brk)
    pltpu.async_copy(xt_hbm.at[:, pl.ds(wid, 1)], inb, sin)

    lane = lax.iota(jnp.int32, L)
    ones = jnp.full((L,), 1.0, jnp.float32)
    zeros = jnp.zeros((L,), jnp.float32)
    zi = jnp.zeros((L,), jnp.int32)

    @plsc.parallel_loop(0, CHW // L, unroll=4)
    def _(j):
        w = j * L + lane
        k = w >> 10
        rem = w & 1023
        fl = rem >> 7
        rl = rem & 127
        plsc.store_scatter(st0, [k, zi, zi, fl, rl], zeros)
        plsc.store_scatter(st1, [k, zi, zi, fl, rl], zeros)

    pltpu.make_async_copy(xt_hbm.at[:, pl.ds(0, 1)], inb, sin).wait()

    def unzero(stb, posb):
        @plsc.parallel_loop(0, GROUPS, unroll=4)
        def _(j):
            fl = (j >> 3) + zi
            rl = (j & 7) * L + lane
            g = posb[pl.ds(j * L, L)]
            plsc.store_scatter(stb, [g, zi, zi, fl, rl], zeros)

    def fill(stb, posb, tf):
        @plsc.parallel_loop(0, GROUPS, unroll=4)
        def _(j):
            x = inb[tf, 0, j >> 3, pl.ds((j & 7) * L, L)]
            g = (x * jnp.float32(NBREAKS - 3)).astype(jnp.int32) + 1
            g = jnp.clip(g, 0, NBINS - 1)
            bu = plsc.load_gather(brk, [g + 1])
            bl = plsc.load_gather(brk, [g])
            g = g + (bu <= x).astype(jnp.int32) - (bl > x).astype(jnp.int32)
            fl = (j >> 3) + zi
            rl = (j & 7) * L + lane
            plsc.store_scatter(stb, [g, zi, zi, fl, rl], ones)
            posb[pl.ds(j * L, L)] = g

    bufs = ((st0, pos0, sout0), (st1, pos1, sout1))

    def chunk_pair(p, c):
        for b in range(2):
            stb, posb, soutb = bufs[b]
            tf = 2 * p + b

            @pl.when(tf >= 2)
            def _():
                pltpu.make_async_copy(
                    stb, out5_hbm.at[:, pl.ds(0, 1), pl.ds(0, 1)], soutb
                ).wait()
                unzero(stb, posb)

            fill(stb, posb, tf)

            pltpu.async_copy(
                stb, out5_hbm.at[:, pl.ds(tf, 1), pl.ds(wid, 1)], soutb
            )

        return c

    lax.fori_loop(0, TF // 2, chunk_pair, 0)

    pltpu.make_async_copy(
        st0, out5_hbm.at[:, pl.ds(0, 1), pl.ds(0, 1)], sout0
    ).wait()
    unzero(st0, pos0)
    fill(st0, pos0, TF - 1)
    pltpu.async_copy(
        st0, out5_hbm.at[:, pl.ds(TF - 1, 1), pl.ds(wid, 1)], sout0
    )
    pltpu.make_async_copy(
        st0, out5_hbm.at[:, pl.ds(0, 1), pl.ds(0, 1)], sout0
    ).wait()
    pltpu.make_async_copy(
        st1, out5_hbm.at[:, pl.ds(0, 1), pl.ds(0, 1)], sout1
    ).wait()


def kernel(input, breaks):
    pad = jnp.full((BPAD - NBREAKS,), jnp.inf, dtype=breaks.dtype)
    breaks_padded = jnp.concatenate([breaks, pad])
    xt = input.reshape(TR, 128, TF, 8).transpose(2, 0, 3, 1)
    out5 = _sc_onehot(xt, breaks_padded)
    return out5.transpose(2, 4, 1, 3, 0).reshape(ROWS, COLS, NBINS)
